# baseline (speedup 1.0000x reference)
import jax
import jax.numpy as jnp
from jax import lax
from jax.experimental import pallas as pl
from jax.experimental.pallas import tpu as pltpu
from jax.experimental.pallas import tpu_sc as plsc

DIM = 64
WIDE = 80
IWORDS = 40
AJWORD = 32
AIW = 16
CHUNK = 128
NBUF = 2
NCORES = 2
NSUB = 16
NW = NCORES * NSUB
LANES = 16


def _tc1_body(x_ref, emb_ref, wlin_ref, wv_ref, bv_ref, ati_ref, atj_ref,
              atei_ref, atej_ref, xlp_ref, xlf_ref, tout_ref, ai_ref, se_ref):
    x = x_ref[...]
    e = emb_ref[...]
    xl = jnp.dot(x, wlin_ref[...], preferred_element_type=jnp.float32)
    tout_ref[...] = (
        jnp.dot(x, wv_ref[...], preferred_element_type=jnp.float32) + bv_ref[...]
    )
    ai = (jnp.dot(xl, ati_ref[...], preferred_element_type=jnp.float32)
          + jnp.dot(e, atei_ref[...], preferred_element_type=jnp.float32))
    aj = (jnp.dot(xl, atj_ref[...], preferred_element_type=jnp.float32)
          + jnp.dot(e, atej_ref[...], preferred_element_type=jnp.float32))
    blk = x.shape[0]
    ai_ref[...] = jnp.concatenate(
        [ai, jnp.zeros((blk, AIW - 1), jnp.float32)], axis=1)
    z = ai + aj
    se_ref[...] = jnp.exp(jnp.maximum(z, 0.2 * z))
    xl_bf = xl.astype(jnp.bfloat16)
    inter = jnp.stack([xl_bf[:, :DIM // 2], xl_bf[:, DIM // 2:]],
                      axis=2).reshape(blk, DIM)
    xlp_ref[...] = jnp.concatenate(
        [inter, aj.astype(jnp.bfloat16),
         jnp.zeros((blk, 2 * IWORDS - DIM - 1), jnp.bfloat16)], axis=1)
    xlf_ref[...] = xl


def _sc_body(src_hbm, dst_hbm, ai_hbm, xlp_hbm, zrows_hbm, out_hbm,
             src_v0, src_v1, dst_v0, dst_v1, dsc_v0, dsc_v1,
             rows_v0, rows_v1, aid_v0, aid_v1, rowsf_v, acc_sh,
             semi0, semi1, semg0, semg1, semw0, semw1):
    cid = lax.axis_index("c")
    sid = lax.axis_index("s")
    wid = sid * NCORES + cid

    srcv = (src_v0, src_v1)
    dstv = (dst_v0, dst_v1)
    dscv = (dsc_v0, dsc_v1)
    rowsv = (rows_v0, rows_v1)
    aidv = (aid_v0, aid_v1)
    semi = (semi0, semi1)
    semg = (semg0, semg1)
    semw = (semw0, semw1)

    rows_per_tile = acc_sh.shape[0] // NSUB
    n_edges = src_hbm.shape[0]
    ept = n_edges // NW
    nchunk = ept // CHUNK
    base = wid * ept

    def issue_idx(t, b):
        off = base + t * CHUNK
        pltpu.async_copy(src_hbm.at[pl.ds(off, CHUNK)], srcv[b], semi[b])
        pltpu.async_copy(dst_hbm.at[pl.ds(off, CHUNK)], dstv[b], semi[b])

    def wait_idx(b):
        pltpu.make_async_copy(src_hbm.at[pl.ds(0, CHUNK)], srcv[b], semi[b]).wait()
        pltpu.make_async_copy(dst_hbm.at[pl.ds(0, CHUNK)], dstv[b], semi[b]).wait()

    def issue_gather(b):
        pltpu.async_copy(xlp_hbm.at[srcv[b]], rowsv[b], semg[b])
        pltpu.async_copy(ai_hbm.at[dstv[b]], aidv[b], semg[b])

    def wait_gather(b):
        pltpu.make_async_copy(xlp_hbm.at[srcv[b]], rowsv[b], semg[b]).wait()
        pltpu.make_async_copy(ai_hbm.at[dstv[b]], aidv[b], semg[b]).wait()

    nq = 4
    qrows = CHUNK // nq
    gpq = qrows // LANES

    def issue_scatter_q(b, q):
        pltpu.async_copy(rowsf_v.at[pl.ds(q * qrows, qrows)],
                         acc_sh.at[dscv[b].at[q]], semw[b], add=True)

    def wait_scatter(b):
        for q in range(nq):
            pltpu.make_async_copy(rowsf_v.at[pl.ds(q * qrows, qrows)],
                                  acc_sh.at[dscv[b].at[q]], semw[b]).wait()

    HIMASK = jnp.int32(-65536)

    def compute_group(b, g):
        col = jnp.full((LANES,), AJWORD, jnp.int32)
        zero = jnp.zeros((LANES,), jnp.int32)
        s16 = srcv[b][pl.ds(g * LANES, LANES)]
        d16 = dstv[b][pl.ds(g * LANES, LANES)]
        rid = lax.iota(jnp.int32, LANES) + g * LANES
        dscv[b][g // gpq, pl.ds((g % gpq) * LANES, LANES)] = d16
        ajw = plsc.load_gather(rowsv[b], [rid, col])
        ajs = plsc.bitcast(lax.shift_left(ajw, 16), jnp.float32)
        aid = plsc.load_gather(aidv[b], [rid, zero])
        a = aid + ajs
        a = jnp.maximum(a, 0.2 * a)
        ex = jnp.exp(a)
        ex = jnp.where(s16 != d16, ex, jnp.zeros_like(ex))
        for jj in range(LANES):
            e = ex[jj]
            j = g * LANES + jj
            w0 = rowsv[b][j, pl.ds(0, LANES)]
            w1 = rowsv[b][j, pl.ds(LANES, LANES)]
            rowsf_v[j, pl.ds(0, LANES)] = (
                plsc.bitcast(lax.shift_left(w0, 16), jnp.float32) * e)
            rowsf_v[j, pl.ds(LANES, LANES)] = (
                plsc.bitcast(lax.shift_left(w1, 16), jnp.float32) * e)
            rowsf_v[j, pl.ds(2 * LANES, LANES)] = (
                plsc.bitcast(w0 & HIMASK, jnp.float32) * e)
            rowsf_v[j, pl.ds(3 * LANES, LANES)] = (
                plsc.bitcast(w1 & HIMASK, jnp.float32) * e)
            rowsf_v[j, pl.ds(DIM, LANES)] = jnp.full((LANES,), e, jnp.float32)

    def chunk_body(t, b, first=False, last1=False, last2=False):
        if not last1:
            wait_idx(1 - b)
        if not last1:
            issue_gather(1 - b)
        wait_gather(b)
        if not first:
            wait_scatter(1 - b)
        for q in range(nq):
            for gg in range(gpq):
                compute_group(b, q * gpq + gg)
            issue_scatter_q(b, q)
        if not (last1 or last2):
            issue_idx(t + 2, b)

    pltpu.sync_copy(zrows_hbm, acc_sh.at[pl.ds(sid * rows_per_tile, rows_per_tile)])
    plsc.subcore_barrier()

    issue_idx(0, 0)
    issue_idx(1, 1)
    wait_idx(0)
    issue_gather(0)

    chunk_body(0, 0, first=True)
    chunk_body(1, 1)

    @pl.loop(0, (nchunk - 4) // 2)
    def _pair(kk):
        t0 = 2 + kk * 2
        chunk_body(t0, 0)
        chunk_body(t0 + 1, 1)

    chunk_body(nchunk - 2, 0, last2=True)
    chunk_body(nchunk - 1, 1, last1=True)
    wait_scatter(1)

    plsc.subcore_barrier()
    sl = pl.ds(sid * rows_per_tile, rows_per_tile)
    pltpu.sync_copy(acc_sh.at[sl], out_hbm.at[cid].at[sl])


def _tc23_body(a0_ref, a1_ref, xlf_ref, se_ref, bias_ref, tout_ref,
               gam_ref, bet_ref, fw1_ref, fb1_ref, fw2_ref, fb2_ref,
               ow_ref, ob_ref, bn_ref, o_ref, outg_s, stats_s):
    p = pl.program_id(0)
    i = pl.program_id(1)
    blk = tout_ref.shape[0]

    @pl.when(p == 0)
    def _():
        a0 = a0_ref[...]
        a1 = a1_ref[...]
        xlf = xlf_ref[...]
        se = se_ref[...]
        num = a0[:, :DIM] + a1[:, :DIM] + se * xlf
        den = a0[:, DIM:DIM + 1] + a1[:, DIM:DIM + 1] + se
        outg = num / (den + 1e-16) + bias_ref[...]
        outg_s[pl.ds(i * blk, blk), :] = outg
        s1 = jnp.sum(outg, axis=0, keepdims=True)
        s2 = jnp.sum(outg * outg, axis=0, keepdims=True)
        st = jnp.concatenate([s1, s2, jnp.zeros((6, DIM), jnp.float32)], axis=0)

        @pl.when(i == 0)
        def _():
            stats_s[...] = st

        @pl.when(i != 0)
        def _():
            stats_s[...] = stats_s[...] + st

    @pl.when(p == 1)
    def _():
        st = stats_s[...]
        inv_n = 1.0 / bn_ref[0, 0]
        mu = st[0:1, :] * inv_n
        var = st[1:2, :] * inv_n - mu * mu
        inv = lax.rsqrt(var + 1e-5)
        outg = outg_s[pl.ds(i * blk, blk), :]
        sp = (outg - mu) * inv * gam_ref[...] + bet_ref[...]
        sp = jnp.maximum(sp, 0.0)
        comb = jnp.concatenate([sp, tout_ref[...]], axis=1)
        h = (jnp.dot(comb, fw1_ref[...], preferred_element_type=jnp.float32)
             + fb1_ref[...])
        h = jnp.maximum(h, 0.0)
        f = (jnp.dot(h, fw2_ref[...], preferred_element_type=jnp.float32)
             + fb2_ref[...])
        o_ref[...] = (jnp.dot(f, ow_ref[...], preferred_element_type=jnp.float32)
                      + ob_ref[...])


def kernel(data, edge_index, emb, Wq, bq, Wk, bk, Wv, bv, temp, W_lin,
           att_i, att_j, att_em_i, att_em_j, gnn_bias, bn_gamma, bn_beta,
           fW1, fb1, fW2, fb2, oW, ob):
    Bsz, N, S = data.shape
    BN = Bsz * N
    E = edge_index.shape[1]
    x = data.reshape(BN, S)

    src = jnp.concatenate([edge_index[0] + i * N for i in range(Bsz)])
    dst = jnp.concatenate([edge_index[1] + i * N for i in range(Bsz)])
    n_edges = Bsz * E
    ept = -(-n_edges // (NW * NBUF * CHUNK)) * (NBUF * CHUNK)
    pad = NW * ept - n_edges
    if pad:
        src = jnp.concatenate([src, jnp.zeros((pad,), jnp.int32)])
        dst = jnp.concatenate([dst, jnp.zeros((pad,), jnp.int32)])

    BLK = 2000
    grid = (BN // BLK,)
    nblk_emb = N // BLK

    full = lambda shape: pl.BlockSpec(shape, lambda b: (0, 0))
    xlp, xlf, tout, ai16, se = pl.pallas_call(
        _tc1_body,
        grid=grid,
        in_specs=[
            pl.BlockSpec((BLK, DIM), lambda b: (b, 0)),
            pl.BlockSpec((BLK, DIM), lambda b: (b % nblk_emb, 0)),
            full((DIM, DIM)), full((DIM, DIM)), full((1, DIM)),
            full((DIM, 1)), full((DIM, 1)), full((DIM, 1)), full((DIM, 1)),
        ],
        out_specs=[
            pl.BlockSpec((BLK, 2 * IWORDS), lambda b: (b, 0)),
            pl.BlockSpec((BLK, DIM), lambda b: (b, 0)),
            pl.BlockSpec((BLK, DIM), lambda b: (b, 0)),
            pl.BlockSpec((BLK, AIW), lambda b: (b, 0)),
            pl.BlockSpec((BLK, 1), lambda b: (b, 0)),
        ],
        out_shape=[
            jax.ShapeDtypeStruct((BN, 2 * IWORDS), jnp.bfloat16),
            jax.ShapeDtypeStruct((BN, DIM), jnp.float32),
            jax.ShapeDtypeStruct((BN, DIM), jnp.float32),
            jax.ShapeDtypeStruct((BN, AIW), jnp.float32),
            jax.ShapeDtypeStruct((BN, 1), jnp.float32),
        ],
    )(x, emb, W_lin, Wv, bv.reshape(1, DIM),
      att_i.reshape(DIM, 1), att_j.reshape(DIM, 1),
      att_em_i.reshape(DIM, 1), att_em_j.reshape(DIM, 1))

    BNP = -(-BN // (8 * NSUB)) * (8 * NSUB)
    zrows = jnp.zeros((BNP // NSUB, WIDE), jnp.float32)
    mesh = plsc.VectorSubcoreMesh(core_axis_name="c", subcore_axis_name="s",
                                  num_cores=NCORES, num_subcores=NSUB)
    acc = pl.kernel(
        _sc_body,
        out_type=jax.ShapeDtypeStruct((NCORES, BNP, WIDE), jnp.float32),
        mesh=mesh,
        compiler_params=pltpu.CompilerParams(needs_layout_passes=False,
                                             use_tc_tiling_on_sc=False),
        scratch_types=[
            pltpu.VMEM((CHUNK,), jnp.int32),
            pltpu.VMEM((CHUNK,), jnp.int32),
            pltpu.VMEM((CHUNK,), jnp.int32),
            pltpu.VMEM((CHUNK,), jnp.int32),
            pltpu.VMEM((4, CHUNK // 4), jnp.int32),
            pltpu.VMEM((4, CHUNK // 4), jnp.int32),
            pltpu.VMEM((CHUNK, IWORDS), jnp.int32),
            pltpu.VMEM((CHUNK, IWORDS), jnp.int32),
            pltpu.VMEM((CHUNK, AIW), jnp.float32),
            pltpu.VMEM((CHUNK, AIW), jnp.float32),
            pltpu.VMEM((CHUNK, WIDE), jnp.float32),
            pltpu.VMEM_SHARED((BNP, WIDE), jnp.float32),
            pltpu.SemaphoreType.DMA,
            pltpu.SemaphoreType.DMA,
            pltpu.SemaphoreType.DMA,
            pltpu.SemaphoreType.DMA,
            pltpu.SemaphoreType.DMA,
            pltpu.SemaphoreType.DMA,
        ],
    )(src, dst, ai16,
      lax.bitcast_convert_type(xlp.reshape(BN, IWORDS, 2), jnp.int32), zrows)

    full2 = lambda shape: pl.BlockSpec(shape, lambda p, b: (0, 0))
    o = pl.pallas_call(
        _tc23_body,
        grid=(2,) + grid,
        in_specs=[
            pl.BlockSpec((BLK, WIDE), lambda p, b: (b * (1 - p), 0)),
            pl.BlockSpec((BLK, WIDE), lambda p, b: (b * (1 - p), 0)),
            pl.BlockSpec((BLK, DIM), lambda p, b: (b * (1 - p), 0)),
            pl.BlockSpec((BLK, 1), lambda p, b: (b * (1 - p), 0)),
            full2((1, DIM)),
            pl.BlockSpec((BLK, DIM), lambda p, b: (b * p, 0)),
            full2((1, DIM)), full2((1, DIM)),
            full2((2 * DIM, DIM)), full2((1, DIM)),
            full2((DIM, DIM)), full2((1, DIM)),
            full2((DIM, 1)), full2((1, 1)), full2((1, 1)),
        ],
        out_specs=pl.BlockSpec((BLK, 1), lambda p, b: (b, 0)),
        out_shape=jax.ShapeDtypeStruct((BN, 1), jnp.float32),
        scratch_shapes=[
            pltpu.VMEM((BN, DIM), jnp.float32),
            pltpu.VMEM((8, DIM), jnp.float32),
        ],
    )(acc[0], acc[1], xlf, se, gnn_bias.reshape(1, DIM),
      tout, bn_gamma.reshape(1, DIM), bn_beta.reshape(1, DIM),
      fW1, fb1.reshape(1, DIM), fW2, fb2.reshape(1, DIM),
      oW, ob.reshape(1, 1), jnp.full((1, 1), float(BN), jnp.float32))

    return o.reshape(BN)


# speedup vs baseline: 1.0084x; 1.0084x over previous
import jax
import jax.numpy as jnp
from jax import lax
from jax.experimental import pallas as pl
from jax.experimental.pallas import tpu as pltpu
from jax.experimental.pallas import tpu_sc as plsc

DIM = 64
WIDE = 80
IWORDS = 48
AJWORD = 32
AIW = 16
CHUNK = 128
NBUF = 2
NCORES = 2
NSUB = 16
NW = NCORES * NSUB
LANES = 16


def _tc1_body(x_ref, emb_ref, wlin_ref, wv_ref, bv_ref, ati_ref, atj_ref,
              atei_ref, atej_ref, xlp_ref, xlf_ref, tout_ref, ai_ref, se_ref):
    x = x_ref[...]
    e = emb_ref[...]
    xl = jnp.dot(x, wlin_ref[...], preferred_element_type=jnp.float32)
    tout_ref[...] = (
        jnp.dot(x, wv_ref[...], preferred_element_type=jnp.float32) + bv_ref[...]
    )
    ai = (jnp.dot(xl, ati_ref[...], preferred_element_type=jnp.float32)
          + jnp.dot(e, atei_ref[...], preferred_element_type=jnp.float32))
    aj = (jnp.dot(xl, atj_ref[...], preferred_element_type=jnp.float32)
          + jnp.dot(e, atej_ref[...], preferred_element_type=jnp.float32))
    blk = x.shape[0]
    ai_ref[...] = jnp.concatenate(
        [ai, jnp.zeros((blk, AIW - 1), jnp.float32)], axis=1)
    z = ai + aj
    se_ref[...] = jnp.exp(jnp.maximum(z, 0.2 * z))
    xl_bf = xl.astype(jnp.bfloat16)
    inter = jnp.stack([xl_bf[:, :DIM // 2], xl_bf[:, DIM // 2:]],
                      axis=2).reshape(blk, DIM)
    xlp_ref[...] = jnp.concatenate(
        [inter, aj.astype(jnp.bfloat16),
         jnp.zeros((blk, 2 * IWORDS - DIM - 1), jnp.bfloat16)], axis=1)
    xlf_ref[...] = xl


def _sc_body(src_hbm, dst_hbm, ai_hbm, xlp_hbm, zrows_hbm, out_hbm,
             src_v0, src_v1, dst_v0, dst_v1, dsc_v0, dsc_v1,
             rows_v0, rows_v1, aid_v0, aid_v1, rowsf_v, acc_sh,
             semi0, semi1, semg0, semg1, semw0, semw1):
    cid = lax.axis_index("c")
    sid = lax.axis_index("s")
    wid = sid * NCORES + cid

    srcv = (src_v0, src_v1)
    dstv = (dst_v0, dst_v1)
    dscv = (dsc_v0, dsc_v1)
    rowsv = (rows_v0, rows_v1)
    aidv = (aid_v0, aid_v1)
    semi = (semi0, semi1)
    semg = (semg0, semg1)
    semw = (semw0, semw1)

    rows_per_tile = acc_sh.shape[0] // NSUB
    n_edges = src_hbm.shape[0]
    ept = n_edges // NW
    nchunk = ept // CHUNK
    base = wid * ept

    def issue_idx(t, b):
        off = base + t * CHUNK
        pltpu.async_copy(src_hbm.at[pl.ds(off, CHUNK)], srcv[b], semi[b])
        pltpu.async_copy(dst_hbm.at[pl.ds(off, CHUNK)], dstv[b], semi[b])

    def wait_idx(b):
        pltpu.make_async_copy(src_hbm.at[pl.ds(0, CHUNK)], srcv[b], semi[b]).wait()
        pltpu.make_async_copy(dst_hbm.at[pl.ds(0, CHUNK)], dstv[b], semi[b]).wait()

    def issue_gather(b):
        pltpu.async_copy(xlp_hbm.at[srcv[b]], rowsv[b], semg[b])
        pltpu.async_copy(ai_hbm.at[dstv[b]], aidv[b], semg[b])

    def wait_gather(b):
        pltpu.make_async_copy(xlp_hbm.at[srcv[b]], rowsv[b], semg[b]).wait()
        pltpu.make_async_copy(ai_hbm.at[dstv[b]], aidv[b], semg[b]).wait()

    nq = 4
    qrows = CHUNK // nq
    gpq = qrows // LANES

    def issue_scatter_q(b, q):
        pltpu.async_copy(rowsf_v.at[pl.ds(q * qrows, qrows)],
                         acc_sh.at[dscv[b].at[q]], semw[b], add=True)

    def wait_scatter(b):
        for q in range(nq):
            pltpu.make_async_copy(rowsf_v.at[pl.ds(q * qrows, qrows)],
                                  acc_sh.at[dscv[b].at[q]], semw[b]).wait()

    HIMASK = jnp.int32(-65536)

    def compute_group(b, g):
        col = jnp.full((LANES,), AJWORD, jnp.int32)
        zero = jnp.zeros((LANES,), jnp.int32)
        s16 = srcv[b][pl.ds(g * LANES, LANES)]
        d16 = dstv[b][pl.ds(g * LANES, LANES)]
        rid = lax.iota(jnp.int32, LANES) + g * LANES
        dscv[b][g // gpq, pl.ds((g % gpq) * LANES, LANES)] = d16
        ajw = plsc.load_gather(rowsv[b], [rid, col])
        ajs = plsc.bitcast(lax.shift_left(ajw, 16), jnp.float32)
        aid = plsc.load_gather(aidv[b], [rid, zero])
        a = aid + ajs
        a = jnp.maximum(a, 0.2 * a)
        ex = jnp.exp(a)
        ex = jnp.where(s16 != d16, ex, jnp.zeros_like(ex))
        for jj in range(LANES):
            e = ex[jj]
            j = g * LANES + jj
            w0 = rowsv[b][j, pl.ds(0, LANES)]
            w1 = rowsv[b][j, pl.ds(LANES, LANES)]
            rowsf_v[j, pl.ds(0, LANES)] = (
                plsc.bitcast(lax.shift_left(w0, 16), jnp.float32) * e)
            rowsf_v[j, pl.ds(LANES, LANES)] = (
                plsc.bitcast(lax.shift_left(w1, 16), jnp.float32) * e)
            rowsf_v[j, pl.ds(2 * LANES, LANES)] = (
                plsc.bitcast(w0 & HIMASK, jnp.float32) * e)
            rowsf_v[j, pl.ds(3 * LANES, LANES)] = (
                plsc.bitcast(w1 & HIMASK, jnp.float32) * e)
            rowsf_v[j, pl.ds(DIM, LANES)] = jnp.full((LANES,), e, jnp.float32)

    def chunk_body(t, b, first=False, last1=False, last2=False):
        if not last1:
            wait_idx(1 - b)
        if not last1:
            issue_gather(1 - b)
        wait_gather(b)
        if not first:
            wait_scatter(1 - b)
        for q in range(nq):
            for gg in range(gpq):
                compute_group(b, q * gpq + gg)
            issue_scatter_q(b, q)
        if not (last1 or last2):
            issue_idx(t + 2, b)

    pltpu.sync_copy(zrows_hbm, acc_sh.at[pl.ds(sid * rows_per_tile, rows_per_tile)])
    plsc.subcore_barrier()

    issue_idx(0, 0)
    issue_idx(1, 1)
    wait_idx(0)
    issue_gather(0)

    chunk_body(0, 0, first=True)
    chunk_body(1, 1)

    @pl.loop(0, (nchunk - 4) // 2)
    def _pair(kk):
        t0 = 2 + kk * 2
        chunk_body(t0, 0)
        chunk_body(t0 + 1, 1)

    chunk_body(nchunk - 2, 0, last2=True)
    chunk_body(nchunk - 1, 1, last1=True)
    wait_scatter(1)

    plsc.subcore_barrier()
    sl = pl.ds(sid * rows_per_tile, rows_per_tile)
    pltpu.sync_copy(acc_sh.at[sl], out_hbm.at[cid].at[sl])


def _tc23_body(a0_ref, a1_ref, xlf_ref, se_ref, bias_ref, tout_ref,
               gam_ref, bet_ref, fw1_ref, fb1_ref, fw2_ref, fb2_ref,
               ow_ref, ob_ref, bn_ref, o_ref, outg_s, stats_s):
    p = pl.program_id(0)
    i = pl.program_id(1)
    blk = tout_ref.shape[0]

    @pl.when(p == 0)
    def _():
        a0 = a0_ref[...]
        a1 = a1_ref[...]
        xlf = xlf_ref[...]
        se = se_ref[...]
        num = a0[:, :DIM] + a1[:, :DIM] + se * xlf
        den = a0[:, DIM:DIM + 1] + a1[:, DIM:DIM + 1] + se
        outg = num / (den + 1e-16) + bias_ref[...]
        outg_s[pl.ds(i * blk, blk), :] = outg
        s1 = jnp.sum(outg, axis=0, keepdims=True)
        s2 = jnp.sum(outg * outg, axis=0, keepdims=True)
        st = jnp.concatenate([s1, s2, jnp.zeros((6, DIM), jnp.float32)], axis=0)

        @pl.when(i == 0)
        def _():
            stats_s[...] = st

        @pl.when(i != 0)
        def _():
            stats_s[...] = stats_s[...] + st

    @pl.when(p == 1)
    def _():
        st = stats_s[...]
        inv_n = 1.0 / bn_ref[0, 0]
        mu = st[0:1, :] * inv_n
        var = st[1:2, :] * inv_n - mu * mu
        inv = lax.rsqrt(var + 1e-5)
        outg = outg_s[pl.ds(i * blk, blk), :]
        sp = (outg - mu) * inv * gam_ref[...] + bet_ref[...]
        sp = jnp.maximum(sp, 0.0)
        comb = jnp.concatenate([sp, tout_ref[...]], axis=1)
        h = (jnp.dot(comb, fw1_ref[...], preferred_element_type=jnp.float32)
             + fb1_ref[...])
        h = jnp.maximum(h, 0.0)
        f = (jnp.dot(h, fw2_ref[...], preferred_element_type=jnp.float32)
             + fb2_ref[...])
        o_ref[...] = (jnp.dot(f, ow_ref[...], preferred_element_type=jnp.float32)
                      + ob_ref[...])


def kernel(data, edge_index, emb, Wq, bq, Wk, bk, Wv, bv, temp, W_lin,
           att_i, att_j, att_em_i, att_em_j, gnn_bias, bn_gamma, bn_beta,
           fW1, fb1, fW2, fb2, oW, ob):
    Bsz, N, S = data.shape
    BN = Bsz * N
    E = edge_index.shape[1]
    x = data.reshape(BN, S)

    src = jnp.concatenate([edge_index[0] + i * N for i in range(Bsz)])
    dst = jnp.concatenate([edge_index[1] + i * N for i in range(Bsz)])
    n_edges = Bsz * E
    ept = -(-n_edges // (NW * NBUF * CHUNK)) * (NBUF * CHUNK)
    pad = NW * ept - n_edges
    if pad:
        src = jnp.concatenate([src, jnp.zeros((pad,), jnp.int32)])
        dst = jnp.concatenate([dst, jnp.zeros((pad,), jnp.int32)])

    BLK = 2000
    grid = (BN // BLK,)
    nblk_emb = N // BLK

    full = lambda shape: pl.BlockSpec(shape, lambda b: (0, 0))
    xlp, xlf, tout, ai16, se = pl.pallas_call(
        _tc1_body,
        grid=grid,
        in_specs=[
            pl.BlockSpec((BLK, DIM), lambda b: (b, 0)),
            pl.BlockSpec((BLK, DIM), lambda b: (b % nblk_emb, 0)),
            full((DIM, DIM)), full((DIM, DIM)), full((1, DIM)),
            full((DIM, 1)), full((DIM, 1)), full((DIM, 1)), full((DIM, 1)),
        ],
        out_specs=[
            pl.BlockSpec((BLK, 2 * IWORDS), lambda b: (b, 0)),
            pl.BlockSpec((BLK, DIM), lambda b: (b, 0)),
            pl.BlockSpec((BLK, DIM), lambda b: (b, 0)),
            pl.BlockSpec((BLK, AIW), lambda b: (b, 0)),
            pl.BlockSpec((BLK, 1), lambda b: (b, 0)),
        ],
        out_shape=[
            jax.ShapeDtypeStruct((BN, 2 * IWORDS), jnp.bfloat16),
            jax.ShapeDtypeStruct((BN, DIM), jnp.float32),
            jax.ShapeDtypeStruct((BN, DIM), jnp.float32),
            jax.ShapeDtypeStruct((BN, AIW), jnp.float32),
            jax.ShapeDtypeStruct((BN, 1), jnp.float32),
        ],
    )(x, emb, W_lin, Wv, bv.reshape(1, DIM),
      att_i.reshape(DIM, 1), att_j.reshape(DIM, 1),
      att_em_i.reshape(DIM, 1), att_em_j.reshape(DIM, 1))

    BNP = -(-BN // (8 * NSUB)) * (8 * NSUB)
    zrows = jnp.zeros((BNP // NSUB, WIDE), jnp.float32)
    mesh = plsc.VectorSubcoreMesh(core_axis_name="c", subcore_axis_name="s",
                                  num_cores=NCORES, num_subcores=NSUB)
    acc = pl.kernel(
        _sc_body,
        out_type=jax.ShapeDtypeStruct((NCORES, BNP, WIDE), jnp.float32),
        mesh=mesh,
        compiler_params=pltpu.CompilerParams(needs_layout_passes=False,
                                             use_tc_tiling_on_sc=False),
        scratch_types=[
            pltpu.VMEM((CHUNK,), jnp.int32),
            pltpu.VMEM((CHUNK,), jnp.int32),
            pltpu.VMEM((CHUNK,), jnp.int32),
            pltpu.VMEM((CHUNK,), jnp.int32),
            pltpu.VMEM((4, CHUNK // 4), jnp.int32),
            pltpu.VMEM((4, CHUNK // 4), jnp.int32),
            pltpu.VMEM((CHUNK, IWORDS), jnp.int32),
            pltpu.VMEM((CHUNK, IWORDS), jnp.int32),
            pltpu.VMEM((CHUNK, AIW), jnp.float32),
            pltpu.VMEM((CHUNK, AIW), jnp.float32),
            pltpu.VMEM((CHUNK, WIDE), jnp.float32),
            pltpu.VMEM_SHARED((BNP, WIDE), jnp.float32),
            pltpu.SemaphoreType.DMA,
            pltpu.SemaphoreType.DMA,
            pltpu.SemaphoreType.DMA,
            pltpu.SemaphoreType.DMA,
            pltpu.SemaphoreType.DMA,
            pltpu.SemaphoreType.DMA,
        ],
    )(src, dst, ai16,
      lax.bitcast_convert_type(xlp.reshape(BN, IWORDS, 2), jnp.int32), zrows)

    full2 = lambda shape: pl.BlockSpec(shape, lambda p, b: (0, 0))
    o = pl.pallas_call(
        _tc23_body,
        grid=(2,) + grid,
        in_specs=[
            pl.BlockSpec((BLK, WIDE), lambda p, b: (b * (1 - p), 0)),
            pl.BlockSpec((BLK, WIDE), lambda p, b: (b * (1 - p), 0)),
            pl.BlockSpec((BLK, DIM), lambda p, b: (b * (1 - p), 0)),
            pl.BlockSpec((BLK, 1), lambda p, b: (b * (1 - p), 0)),
            full2((1, DIM)),
            pl.BlockSpec((BLK, DIM), lambda p, b: (b * p, 0)),
            full2((1, DIM)), full2((1, DIM)),
            full2((2 * DIM, DIM)), full2((1, DIM)),
            full2((DIM, DIM)), full2((1, DIM)),
            full2((DIM, 1)), full2((1, 1)), full2((1, 1)),
        ],
        out_specs=pl.BlockSpec((BLK, 1), lambda p, b: (b, 0)),
        out_shape=jax.ShapeDtypeStruct((BN, 1), jnp.float32),
        scratch_shapes=[
            pltpu.VMEM((BN, DIM), jnp.float32),
            pltpu.VMEM((8, DIM), jnp.float32),
        ],
    )(acc[0], acc[1], xlf, se, gnn_bias.reshape(1, DIM),
      tout, bn_gamma.reshape(1, DIM), bn_beta.reshape(1, DIM),
      fW1, fb1.reshape(1, DIM), fW2, fb2.reshape(1, DIM),
      oW, ob.reshape(1, 1), jnp.full((1, 1), float(BN), jnp.float32))

    return o.reshape(BN)


# speedup vs baseline: 1.0782x; 1.0692x over previous
import jax
import jax.numpy as jnp
from jax import lax
from jax.experimental import pallas as pl
from jax.experimental.pallas import tpu as pltpu
from jax.experimental.pallas import tpu_sc as plsc

DIM = 64
WIDE = 80
AJCOL = 65
AIW = 16
CHUNK = 128
NBUF = 2
NCORES = 2
NSUB = 16
NW = NCORES * NSUB
LANES = 16


def _tc1_body(x_ref, emb_ref, wlin_ref, wv_ref, bv_ref, ati_ref, atj_ref,
              atei_ref, atej_ref, xlp_ref, tout_ref, ai_ref, se_ref):
    x = x_ref[...]
    e = emb_ref[...]
    xl = jnp.dot(x, wlin_ref[...], preferred_element_type=jnp.float32)
    tout_ref[...] = (
        jnp.dot(x, wv_ref[...], preferred_element_type=jnp.float32) + bv_ref[...]
    )
    ai = (jnp.dot(xl, ati_ref[...], preferred_element_type=jnp.float32)
          + jnp.dot(e, atei_ref[...], preferred_element_type=jnp.float32))
    aj = (jnp.dot(xl, atj_ref[...], preferred_element_type=jnp.float32)
          + jnp.dot(e, atej_ref[...], preferred_element_type=jnp.float32))
    blk = x.shape[0]
    ai_ref[...] = jnp.concatenate(
        [ai, jnp.zeros((blk, AIW - 1), jnp.float32)], axis=1)
    z = ai + aj
    se_ref[...] = jnp.exp(jnp.maximum(z, 0.2 * z))
    xlp_ref[...] = jnp.concatenate(
        [xl, jnp.ones((blk, 1), jnp.float32), aj,
         jnp.zeros((blk, WIDE - DIM - 2), jnp.float32)], axis=1)


def _sc_body(src_hbm, dst_hbm, ai_hbm, xlp_hbm, zrows_hbm, out_hbm,
             src_v0, src_v1, dst_v0, dst_v1, dsc_v0, dsc_v1,
             rows_v0, rows_v1, aid_v0, aid_v1, acc_sh,
             semi0, semi1, semg0, semg1, semw0, semw1):
    cid = lax.axis_index("c")
    sid = lax.axis_index("s")
    wid = sid * NCORES + cid

    srcv = (src_v0, src_v1)
    dstv = (dst_v0, dst_v1)
    dscv = (dsc_v0, dsc_v1)
    rowsv = (rows_v0, rows_v1)
    aidv = (aid_v0, aid_v1)
    semi = (semi0, semi1)
    semg = (semg0, semg1)
    semw = (semw0, semw1)

    rows_per_tile = acc_sh.shape[0] // NSUB
    n_edges = src_hbm.shape[0]
    ept = n_edges // NW
    nchunk = ept // CHUNK
    base = wid * ept

    def issue_idx(t, b):
        off = base + t * CHUNK
        pltpu.async_copy(src_hbm.at[pl.ds(off, CHUNK)], srcv[b], semi[b])
        pltpu.async_copy(dst_hbm.at[pl.ds(off, CHUNK)], dstv[b], semi[b])

    def wait_idx(b):
        pltpu.make_async_copy(src_hbm.at[pl.ds(0, CHUNK)], srcv[b], semi[b]).wait()
        pltpu.make_async_copy(dst_hbm.at[pl.ds(0, CHUNK)], dstv[b], semi[b]).wait()

    H = CHUNK // 2

    def issue_gather(b):
        pltpu.async_copy(xlp_hbm.at[srcv[b].at[pl.ds(0, H)]],
                         rowsv[b].at[pl.ds(0, H)], semg[b])
        pltpu.async_copy(xlp_hbm.at[srcv[b].at[pl.ds(H, H)]],
                         rowsv[b].at[pl.ds(H, H)], semg[b])
        pltpu.async_copy(ai_hbm.at[dstv[b]], aidv[b], semg[b])

    def wait_gather(b):
        pltpu.make_async_copy(xlp_hbm.at[srcv[b].at[pl.ds(0, H)]],
                              rowsv[b].at[pl.ds(0, H)], semg[b]).wait()
        pltpu.make_async_copy(xlp_hbm.at[srcv[b].at[pl.ds(H, H)]],
                              rowsv[b].at[pl.ds(H, H)], semg[b]).wait()
        pltpu.make_async_copy(ai_hbm.at[dstv[b]], aidv[b], semg[b]).wait()

    nq = 4
    qrows = CHUNK // nq
    gpq = qrows // LANES

    def issue_scatter_q(b, q):
        pltpu.async_copy(rowsv[b].at[pl.ds(q * qrows, qrows)],
                         acc_sh.at[dscv[b].at[q]], semw[b], add=True)

    def wait_scatter(b):
        for q in range(nq):
            pltpu.make_async_copy(rowsv[b].at[pl.ds(q * qrows, qrows)],
                                  acc_sh.at[dscv[b].at[q]], semw[b]).wait()

    def compute_group(b, g):
        col = jnp.full((LANES,), AJCOL, jnp.int32)
        zero = jnp.zeros((LANES,), jnp.int32)
        s16 = srcv[b][pl.ds(g * LANES, LANES)]
        d16 = dstv[b][pl.ds(g * LANES, LANES)]
        rid = lax.iota(jnp.int32, LANES) + g * LANES
        dscv[b][g // gpq, pl.ds((g % gpq) * LANES, LANES)] = d16
        ajs = plsc.load_gather(rowsv[b], [rid, col])
        aid = plsc.load_gather(aidv[b], [rid, zero])
        a = aid + ajs
        a = jnp.maximum(a, 0.2 * a)
        ex = jnp.exp(a)
        ex = jnp.where(s16 != d16, ex, jnp.zeros_like(ex))
        for jj in range(LANES):
            e = ex[jj]
            j = g * LANES + jj
            for c in range(DIM // LANES):
                sl = pl.ds(c * LANES, LANES)
                rowsv[b][j, sl] = rowsv[b][j, sl] * e
            rowsv[b][j, pl.ds(DIM, LANES)] = jnp.full((LANES,), e, jnp.float32)

    def chunk_body(t, b, first=False, last1=False, last2=False):
        if not last1:
            wait_idx(1 - b)
        if not first:
            wait_scatter(1 - b)
        if not last1:
            issue_gather(1 - b)
        wait_gather(b)
        for q in range(nq):
            for gg in range(gpq):
                compute_group(b, q * gpq + gg)
            issue_scatter_q(b, q)
        if not (last1 or last2):
            issue_idx(t + 2, b)

    pltpu.sync_copy(zrows_hbm, acc_sh.at[pl.ds(sid * rows_per_tile, rows_per_tile)])
    plsc.subcore_barrier()

    issue_idx(0, 0)
    issue_idx(1, 1)
    wait_idx(0)
    issue_gather(0)

    chunk_body(0, 0, first=True)
    chunk_body(1, 1)

    @pl.loop(0, (nchunk - 4) // 2)
    def _pair(kk):
        t0 = 2 + kk * 2
        chunk_body(t0, 0)
        chunk_body(t0 + 1, 1)

    chunk_body(nchunk - 2, 0, last2=True)
    chunk_body(nchunk - 1, 1, last1=True)
    wait_scatter(1)

    plsc.subcore_barrier()
    sl = pl.ds(sid * rows_per_tile, rows_per_tile)
    pltpu.sync_copy(acc_sh.at[sl], out_hbm.at[cid].at[sl])


def _tc23_body(a0_ref, a1_ref, xlp_ref, se_ref, bias_ref, tout_ref,
               gam_ref, bet_ref, fw1_ref, fb1_ref, fw2_ref, fb2_ref,
               ow_ref, ob_ref, bn_ref, o_ref, outg_s, stats_s):
    p = pl.program_id(0)
    i = pl.program_id(1)
    blk = tout_ref.shape[0]

    @pl.when(p == 0)
    def _():
        a0 = a0_ref[...]
        a1 = a1_ref[...]
        xlp = xlp_ref[...]
        se = se_ref[...]
        num = a0[:, :DIM] + a1[:, :DIM] + se * xlp[:, :DIM]
        den = a0[:, DIM:DIM + 1] + a1[:, DIM:DIM + 1] + se
        outg = num / (den + 1e-16) + bias_ref[...]
        outg_s[pl.ds(i * blk, blk), :] = outg
        s1 = jnp.sum(outg, axis=0, keepdims=True)
        s2 = jnp.sum(outg * outg, axis=0, keepdims=True)
        st = jnp.concatenate([s1, s2, jnp.zeros((6, DIM), jnp.float32)], axis=0)

        @pl.when(i == 0)
        def _():
            stats_s[...] = st

        @pl.when(i != 0)
        def _():
            stats_s[...] = stats_s[...] + st

    @pl.when(p == 1)
    def _():
        st = stats_s[...]
        inv_n = 1.0 / bn_ref[0, 0]
        mu = st[0:1, :] * inv_n
        var = st[1:2, :] * inv_n - mu * mu
        inv = lax.rsqrt(var + 1e-5)
        outg = outg_s[pl.ds(i * blk, blk), :]
        sp = (outg - mu) * inv * gam_ref[...] + bet_ref[...]
        sp = jnp.maximum(sp, 0.0)
        comb = jnp.concatenate([sp, tout_ref[...]], axis=1)
        h = (jnp.dot(comb, fw1_ref[...], preferred_element_type=jnp.float32)
             + fb1_ref[...])
        h = jnp.maximum(h, 0.0)
        f = (jnp.dot(h, fw2_ref[...], preferred_element_type=jnp.float32)
             + fb2_ref[...])
        o_ref[...] = (jnp.dot(f, ow_ref[...], preferred_element_type=jnp.float32)
                      + ob_ref[...])


def kernel(data, edge_index, emb, Wq, bq, Wk, bk, Wv, bv, temp, W_lin,
           att_i, att_j, att_em_i, att_em_j, gnn_bias, bn_gamma, bn_beta,
           fW1, fb1, fW2, fb2, oW, ob):
    Bsz, N, S = data.shape
    BN = Bsz * N
    E = edge_index.shape[1]
    x = data.reshape(BN, S)

    src = jnp.concatenate([edge_index[0] + i * N for i in range(Bsz)])
    dst = jnp.concatenate([edge_index[1] + i * N for i in range(Bsz)])
    n_edges = Bsz * E
    ept = -(-n_edges // (NW * NBUF * CHUNK)) * (NBUF * CHUNK)
    pad = NW * ept - n_edges
    if pad:
        src = jnp.concatenate([src, jnp.zeros((pad,), jnp.int32)])
        dst = jnp.concatenate([dst, jnp.zeros((pad,), jnp.int32)])

    BLK = 2000
    grid = (BN // BLK,)
    nblk_emb = N // BLK

    full = lambda shape: pl.BlockSpec(shape, lambda b: (0, 0))
    xlp, tout, ai16, se = pl.pallas_call(
        _tc1_body,
        grid=grid,
        in_specs=[
            pl.BlockSpec((BLK, DIM), lambda b: (b, 0)),
            pl.BlockSpec((BLK, DIM), lambda b: (b % nblk_emb, 0)),
            full((DIM, DIM)), full((DIM, DIM)), full((1, DIM)),
            full((DIM, 1)), full((DIM, 1)), full((DIM, 1)), full((DIM, 1)),
        ],
        out_specs=[
            pl.BlockSpec((BLK, WIDE), lambda b: (b, 0)),
            pl.BlockSpec((BLK, DIM), lambda b: (b, 0)),
            pl.BlockSpec((BLK, AIW), lambda b: (b, 0)),
            pl.BlockSpec((BLK, 1), lambda b: (b, 0)),
        ],
        out_shape=[
            jax.ShapeDtypeStruct((BN, WIDE), jnp.float32),
            jax.ShapeDtypeStruct((BN, DIM), jnp.float32),
            jax.ShapeDtypeStruct((BN, AIW), jnp.float32),
            jax.ShapeDtypeStruct((BN, 1), jnp.float32),
        ],
    )(x, emb, W_lin, Wv, bv.reshape(1, DIM),
      att_i.reshape(DIM, 1), att_j.reshape(DIM, 1),
      att_em_i.reshape(DIM, 1), att_em_j.reshape(DIM, 1))

    BNP = -(-BN // (8 * NSUB)) * (8 * NSUB)
    zrows = jnp.zeros((BNP // NSUB, WIDE), jnp.float32)
    mesh = plsc.VectorSubcoreMesh(core_axis_name="c", subcore_axis_name="s",
                                  num_cores=NCORES, num_subcores=NSUB)
    acc = pl.kernel(
        _sc_body,
        out_type=jax.ShapeDtypeStruct((NCORES, BNP, WIDE), jnp.float32),
        mesh=mesh,
        compiler_params=pltpu.CompilerParams(needs_layout_passes=False,
                                             use_tc_tiling_on_sc=False),
        scratch_types=[
            pltpu.VMEM((CHUNK,), jnp.int32),
            pltpu.VMEM((CHUNK,), jnp.int32),
            pltpu.VMEM((CHUNK,), jnp.int32),
            pltpu.VMEM((CHUNK,), jnp.int32),
            pltpu.VMEM((4, CHUNK // 4), jnp.int32),
            pltpu.VMEM((4, CHUNK // 4), jnp.int32),
            pltpu.VMEM((CHUNK, WIDE), jnp.float32),
            pltpu.VMEM((CHUNK, WIDE), jnp.float32),
            pltpu.VMEM((CHUNK, AIW), jnp.float32),
            pltpu.VMEM((CHUNK, AIW), jnp.float32),
            pltpu.VMEM_SHARED((BNP, WIDE), jnp.float32),
            pltpu.SemaphoreType.DMA,
            pltpu.SemaphoreType.DMA,
            pltpu.SemaphoreType.DMA,
            pltpu.SemaphoreType.DMA,
            pltpu.SemaphoreType.DMA,
            pltpu.SemaphoreType.DMA,
        ],
    )(src, dst, ai16, xlp, zrows)

    full2 = lambda shape: pl.BlockSpec(shape, lambda p, b: (0, 0))
    o = pl.pallas_call(
        _tc23_body,
        grid=(2,) + grid,
        in_specs=[
            pl.BlockSpec((BLK, WIDE), lambda p, b: (b * (1 - p), 0)),
            pl.BlockSpec((BLK, WIDE), lambda p, b: (b * (1 - p), 0)),
            pl.BlockSpec((BLK, WIDE), lambda p, b: (b * (1 - p), 0)),
            pl.BlockSpec((BLK, 1), lambda p, b: (b * (1 - p), 0)),
            full2((1, DIM)),
            pl.BlockSpec((BLK, DIM), lambda p, b: (b * p, 0)),
            full2((1, DIM)), full2((1, DIM)),
            full2((2 * DIM, DIM)), full2((1, DIM)),
            full2((DIM, DIM)), full2((1, DIM)),
            full2((DIM, 1)), full2((1, 1)), full2((1, 1)),
        ],
        out_specs=pl.BlockSpec((BLK, 1), lambda p, b: (b, 0)),
        out_shape=jax.ShapeDtypeStruct((BN, 1), jnp.float32),
        scratch_shapes=[
            pltpu.VMEM((BN, DIM), jnp.float32),
            pltpu.VMEM((8, DIM), jnp.float32),
        ],
    )(acc[0], acc[1], xlp, se, gnn_bias.reshape(1, DIM),
      tout, bn_gamma.reshape(1, DIM), bn_beta.reshape(1, DIM),
      fW1, fb1.reshape(1, DIM), fW2, fb2.reshape(1, DIM),
      oW, ob.reshape(1, 1), jnp.full((1, 1), float(BN), jnp.float32))

    return o.reshape(BN)
